# baseline (device time: 90299 ns/iter reference)
import functools

import jax
import jax.numpy as jnp
from jax import lax
from jax.experimental import pallas as pl
from jax.experimental.pallas import tpu as pltpu

N_Y = 4
EPS = 1e-6


def kernel(partial, resid, gamma):
    m, d = resid.shape
    rows = m // N_Y
    x2d = partial.reshape(m, d)
    g2d = gamma.reshape(1, d)

    def body(p_ref, r_ref, g_ref, o_ref, rs_buf, send_sems, recv_sems):
        my_x = lax.axis_index("x")
        my_y = lax.axis_index("y")
        my_z = lax.axis_index("z")
        left = (my_y + N_Y - 1) % N_Y
        right = (my_y + 1) % N_Y

        barrier = pltpu.get_barrier_semaphore()
        for nbr in (left, right):
            pl.semaphore_signal(
                barrier,
                inc=1,
                device_id=(my_x, nbr, my_z),
                device_id_type=pl.DeviceIdType.MESH,
            )
        pl.semaphore_wait(barrier, 2)

        def row_chunk(ref, c):
            return ref.at[pl.ds(c * rows, rows), :]

        for s in range(N_Y - 1):
            c_send = (my_y + 2 * N_Y - s - 1) % N_Y
            c_recv = (my_y + 2 * N_Y - s - 2) % N_Y
            src = row_chunk(p_ref, c_send) if s == 0 else rs_buf.at[s - 1]
            rdma = pltpu.make_async_remote_copy(
                src_ref=src,
                dst_ref=rs_buf.at[s],
                send_sem=send_sems.at[s],
                recv_sem=recv_sems.at[s],
                device_id=(my_x, right, my_z),
                device_id_type=pl.DeviceIdType.MESH,
            )
            rdma.start()
            rdma.wait()
            rs_buf[s, :, :] = rs_buf[s, :, :] + p_ref[
                pl.ds(c_recv * rows, rows), :
            ]

        y = rs_buf[N_Y - 2, :, :] + r_ref[pl.ds(my_y * rows, rows), :]
        ms = jnp.mean(y * y, axis=-1, keepdims=True)
        o_ref[pl.ds(my_y * rows, rows), :] = y * lax.rsqrt(ms + EPS) * g_ref[:, :]

        for t in range(N_Y - 1):
            c_send = (my_y + N_Y - t) % N_Y
            rdma = pltpu.make_async_remote_copy(
                src_ref=row_chunk(o_ref, c_send),
                dst_ref=row_chunk(o_ref, c_send),
                send_sem=send_sems.at[N_Y - 1 + t],
                recv_sem=recv_sems.at[N_Y - 1 + t],
                device_id=(my_x, right, my_z),
                device_id_type=pl.DeviceIdType.MESH,
            )
            rdma.start()
            rdma.wait()

        @functools.partial(
            pl.run_scoped, second_barrier=pltpu.SemaphoreType.REGULAR
        )
        def _(second_barrier):
            for nbr in (left, right):
                pl.semaphore_signal(
                    second_barrier,
                    inc=1,
                    device_id=(my_x, nbr, my_z),
                    device_id_type=pl.DeviceIdType.MESH,
                )
            pl.semaphore_wait(second_barrier, 2)

    params_cls = getattr(pltpu, "CompilerParams", None) or getattr(
        pltpu, "TPUCompilerParams"
    )
    return pl.pallas_call(
        body,
        out_shape=jax.ShapeDtypeStruct((m, d), jnp.float32),
        in_specs=[
            pl.BlockSpec(memory_space=pltpu.VMEM),
            pl.BlockSpec(memory_space=pltpu.VMEM),
            pl.BlockSpec(memory_space=pltpu.VMEM),
        ],
        out_specs=pl.BlockSpec(memory_space=pltpu.VMEM),
        scratch_shapes=[
            pltpu.VMEM((N_Y - 1, rows, d), jnp.float32),
            pltpu.SemaphoreType.DMA((2 * (N_Y - 1),)),
            pltpu.SemaphoreType.DMA((2 * (N_Y - 1),)),
        ],
        compiler_params=params_cls(collective_id=0),
    )(x2d, resid, g2d)
